# baseline (speedup 1.0000x reference)
import functools

import jax
import jax.numpy as jnp
from jax import lax
from jax.experimental import pallas as pl
from jax.experimental.pallas import tpu as pltpu
from jax.experimental.pallas import tpu_sc as plsc


def kernel(x, token_table, pos_table):
    B, S = x.shape
    V, D = token_table.shape
    N = B * S
    L = 16

    info = plsc.get_sparse_core_info()
    NW = info.num_cores * info.num_subcores
    b_per_w = N // NW
    H = b_per_w // 2
    W_PER_ROW = S // b_per_w
    assert N % NW == 0 and D % L == 0 and H <= 128 and H % 8 == 0
    assert S % b_per_w == 0

    mesh = plsc.VectorSubcoreMesh(core_axis_name="c", subcore_axis_name="s")

    @functools.partial(
        pl.kernel,
        mesh=mesh,
        out_type=jax.ShapeDtypeStruct((B, S, D), jnp.float32),
        scratch_types=[
            pltpu.VMEM((b_per_w,), jnp.int32),
            pltpu.VMEM((H, D), jnp.float32),
            pltpu.VMEM((b_per_w, D), jnp.float32),
            pltpu.SemaphoreType.DMA,
            pltpu.SemaphoreType.DMA,
            pltpu.SemaphoreType.DMA,
            pltpu.SemaphoreType.DMA,
            pltpu.SemaphoreType.DMA,
            pltpu.SemaphoreType.DMA,
            pltpu.SemaphoreType.DMA,
        ],
    )
    def sc_kernel(x_hbm, tok_hbm, pos_hbm, out_hbm, idx_v, pos_v, rows_v,
                  sem_i, sem_p0, sem_p1, sem_g0, sem_g1, sem_w0, sem_w1):
        wid = lax.axis_index("s") * info.num_cores + lax.axis_index("c")
        b_idx = wid // W_PER_ROW
        s_base = lax.rem(wid, W_PER_ROW) * b_per_w

        i_cp = pltpu.async_copy(
            x_hbm.at[b_idx, pl.ds(s_base, b_per_w)], idx_v, sem_i)
        i_cp.wait()
        g0 = pltpu.async_copy(
            tok_hbm.at[idx_v.at[pl.ds(0, H)]], rows_v.at[pl.ds(0, H)],
            sem_g0)
        p0 = pltpu.async_copy(
            pos_hbm.at[pl.ds(s_base, H)], pos_v, sem_p0)
        p1 = pltpu.async_copy(
            pos_hbm.at[pl.ds(s_base + H, H)], rows_v.at[pl.ds(H, H)],
            sem_p1)

        p1.wait()
        g1 = pltpu.async_copy(
            tok_hbm.at[idx_v.at[pl.ds(H, H)]], rows_v.at[pl.ds(H, H)],
            sem_g1, add=True)

        p0.wait()
        g0.wait()

        @plsc.parallel_loop(0, H)
        def add0(i):
            for j in range(D // L):
                sl = pl.ds(j * L, L)
                plsc.addupdate(rows_v.at[i, sl], pos_v[i, sl])

        w0 = pltpu.async_copy(
            rows_v.at[pl.ds(0, H)],
            out_hbm.at[b_idx, pl.ds(s_base, H)], sem_w0)
        g1.wait()
        w1 = pltpu.async_copy(
            rows_v.at[pl.ds(H, H)],
            out_hbm.at[b_idx, pl.ds(s_base + H, H)], sem_w1)
        w0.wait()
        w1.wait()

    return sc_kernel(x, token_table, pos_table)


# speedup vs baseline: 1.0281x; 1.0281x over previous
import functools

import jax
import jax.numpy as jnp
from jax import lax
from jax.experimental import pallas as pl
from jax.experimental.pallas import tpu as pltpu
from jax.experimental.pallas import tpu_sc as plsc


def kernel(x, token_table, pos_table):
    B, S = x.shape
    V, D = token_table.shape
    N = B * S
    L = 16

    info = plsc.get_sparse_core_info()
    NC, NS = info.num_cores, info.num_subcores
    NW = NC * NS
    b_per_w = N // NW
    H = b_per_w // 2
    BLK_PER_CORE = NS // B
    POS_PER_CORE = BLK_PER_CORE * b_per_w
    COOP = POS_PER_CORE // NS
    assert N % NW == 0 and D % L == 0 and H <= 128 and H % 8 == 0
    assert NS % B == 0 and S == NC * POS_PER_CORE and COOP % 8 == 0

    mesh = plsc.VectorSubcoreMesh(core_axis_name="c", subcore_axis_name="s")

    @functools.partial(
        pl.kernel,
        mesh=mesh,
        out_type=jax.ShapeDtypeStruct((B, S, D), jnp.float32),
        scratch_types=[
            pltpu.VMEM((b_per_w,), jnp.int32),
            pltpu.VMEM((H, D), jnp.float32),
            pltpu.VMEM((b_per_w, D), jnp.float32),
            pltpu.VMEM_SHARED((POS_PER_CORE, D), jnp.float32),
            pltpu.SemaphoreType.DMA,
            pltpu.SemaphoreType.DMA,
            pltpu.SemaphoreType.DMA,
            pltpu.SemaphoreType.DMA,
            pltpu.SemaphoreType.DMA,
            pltpu.SemaphoreType.DMA,
            pltpu.SemaphoreType.DMA,
        ],
    )
    def sc_kernel(x_hbm, tok_hbm, pos_hbm, out_hbm, idx_v, pos_v, rows_v,
                  pos_sh, sem_i, sem_p0, sem_p1, sem_g0, sem_g1, sem_w0,
                  sem_w1):
        c = lax.axis_index("c")
        s = lax.axis_index("s")
        b_idx = s // BLK_PER_CORE
        blk = lax.rem(s, BLK_PER_CORE)
        s_base = c * POS_PER_CORE + blk * b_per_w

        i_cp = pltpu.async_copy(
            x_hbm.at[b_idx, pl.ds(s_base, b_per_w)], idx_v, sem_i)
        pltpu.sync_copy(
            pos_hbm.at[pl.ds(c * POS_PER_CORE + s * COOP, COOP)],
            pos_sh.at[pl.ds(s * COOP, COOP)])
        i_cp.wait()
        g0 = pltpu.async_copy(
            tok_hbm.at[idx_v.at[pl.ds(0, H)]], rows_v.at[pl.ds(0, H)],
            sem_g0)
        plsc.subcore_barrier()

        p0 = pltpu.async_copy(
            pos_sh.at[pl.ds(blk * b_per_w, H)], pos_v, sem_p0)
        p1 = pltpu.async_copy(
            pos_sh.at[pl.ds(blk * b_per_w + H, H)], rows_v.at[pl.ds(H, H)],
            sem_p1)

        p1.wait()
        g1 = pltpu.async_copy(
            tok_hbm.at[idx_v.at[pl.ds(H, H)]], rows_v.at[pl.ds(H, H)],
            sem_g1, add=True)

        p0.wait()
        g0.wait()
        Q = H // 2

        @plsc.parallel_loop(0, Q)
        def add0a(i):
            for j in range(D // L):
                sl = pl.ds(j * L, L)
                plsc.addupdate(rows_v.at[i, sl], pos_v[i, sl])

        w0a = pltpu.async_copy(
            rows_v.at[pl.ds(0, Q)],
            out_hbm.at[b_idx, pl.ds(s_base, Q)], sem_w0)

        @plsc.parallel_loop(Q, H)
        def add0b(i):
            for j in range(D // L):
                sl = pl.ds(j * L, L)
                plsc.addupdate(rows_v.at[i, sl], pos_v[i, sl])

        w0 = pltpu.async_copy(
            rows_v.at[pl.ds(Q, Q)],
            out_hbm.at[b_idx, pl.ds(s_base + Q, Q)], sem_w0)
        g1.wait()
        w1 = pltpu.async_copy(
            rows_v.at[pl.ds(H, H)],
            out_hbm.at[b_idx, pl.ds(s_base + H, H)], sem_w1)
        w0a.wait()
        w0.wait()
        w1.wait()

    return sc_kernel(x, token_table, pos_table)


# speedup vs baseline: 1.0336x; 1.0054x over previous
import functools

import jax
import jax.numpy as jnp
from jax import lax
from jax.experimental import pallas as pl
from jax.experimental.pallas import tpu as pltpu
from jax.experimental.pallas import tpu_sc as plsc


def kernel(x, token_table, pos_table):
    B, S = x.shape
    V, D = token_table.shape
    N = B * S
    L = 16

    info = plsc.get_sparse_core_info()
    NC, NS = info.num_cores, info.num_subcores
    NW = NC * NS
    b_per_w = N // NW
    H = b_per_w // 2
    BLK_PER_CORE = NS // B
    POS_PER_CORE = BLK_PER_CORE * b_per_w
    COOP = POS_PER_CORE // NS
    assert N % NW == 0 and D % L == 0 and H <= 128 and H % 8 == 0
    assert NS % B == 0 and S == NC * POS_PER_CORE and COOP % 8 == 0

    mesh = plsc.VectorSubcoreMesh(core_axis_name="c", subcore_axis_name="s")

    @functools.partial(
        pl.kernel,
        mesh=mesh,
        out_type=jax.ShapeDtypeStruct((B, S, D), jnp.float32),
        scratch_types=[
            pltpu.VMEM((b_per_w,), jnp.int32),
            pltpu.VMEM((H, D), jnp.float32),
            pltpu.VMEM((b_per_w, D), jnp.float32),
            pltpu.VMEM_SHARED((POS_PER_CORE, D), jnp.float32),
            pltpu.SemaphoreType.DMA,
            pltpu.SemaphoreType.DMA,
            pltpu.SemaphoreType.DMA,
            pltpu.SemaphoreType.DMA,
            pltpu.SemaphoreType.DMA,
            pltpu.SemaphoreType.DMA,
            pltpu.SemaphoreType.DMA,
            pltpu.SemaphoreType.DMA,
        ],
    )
    def sc_kernel(x_hbm, tok_hbm, pos_hbm, out_hbm, idx_v, pos_v, rows_v,
                  pos_sh, sem_i, sem_p0, sem_p1, sem_g0, sem_g1a, sem_g1b,
                  sem_w0, sem_w1):
        c = lax.axis_index("c")
        s = lax.axis_index("s")
        b_idx = s // BLK_PER_CORE
        blk = lax.rem(s, BLK_PER_CORE)
        s_base = c * POS_PER_CORE + blk * b_per_w

        i_cp = pltpu.async_copy(
            x_hbm.at[b_idx, pl.ds(s_base, b_per_w)], idx_v, sem_i)
        pltpu.sync_copy(
            pos_hbm.at[pl.ds(c * POS_PER_CORE + s * COOP, COOP)],
            pos_sh.at[pl.ds(s * COOP, COOP)])
        i_cp.wait()
        g0 = pltpu.async_copy(
            tok_hbm.at[idx_v.at[pl.ds(0, H)]], rows_v.at[pl.ds(0, H)],
            sem_g0)
        plsc.subcore_barrier()

        p0 = pltpu.async_copy(
            pos_sh.at[pl.ds(blk * b_per_w, H)], pos_v, sem_p0)
        p1 = pltpu.async_copy(
            pos_sh.at[pl.ds(blk * b_per_w + H, H)], rows_v.at[pl.ds(H, H)],
            sem_p1)

        QH = H // 2
        p1.wait()
        g1a = pltpu.async_copy(
            tok_hbm.at[idx_v.at[pl.ds(H, QH)]], rows_v.at[pl.ds(H, QH)],
            sem_g1a, add=True)
        g1b = pltpu.async_copy(
            tok_hbm.at[idx_v.at[pl.ds(H + QH, QH)]],
            rows_v.at[pl.ds(H + QH, QH)], sem_g1b, add=True)

        p0.wait()
        g0.wait()
        Q = H // 2

        @plsc.parallel_loop(0, Q)
        def add0a(i):
            for j in range(D // L):
                sl = pl.ds(j * L, L)
                plsc.addupdate(rows_v.at[i, sl], pos_v[i, sl])

        w0a = pltpu.async_copy(
            rows_v.at[pl.ds(0, Q)],
            out_hbm.at[b_idx, pl.ds(s_base, Q)], sem_w0)

        @plsc.parallel_loop(Q, H)
        def add0b(i):
            for j in range(D // L):
                sl = pl.ds(j * L, L)
                plsc.addupdate(rows_v.at[i, sl], pos_v[i, sl])

        w0 = pltpu.async_copy(
            rows_v.at[pl.ds(Q, Q)],
            out_hbm.at[b_idx, pl.ds(s_base + Q, Q)], sem_w0)
        g1a.wait()
        w1a = pltpu.async_copy(
            rows_v.at[pl.ds(H, QH)],
            out_hbm.at[b_idx, pl.ds(s_base + H, QH)], sem_w1)
        g1b.wait()
        w1b = pltpu.async_copy(
            rows_v.at[pl.ds(H + QH, QH)],
            out_hbm.at[b_idx, pl.ds(s_base + H + QH, QH)], sem_w1)
        w0a.wait()
        w0.wait()
        w1a.wait()
        w1b.wait()

    return sc_kernel(x, token_table, pos_table)
